# baseline (speedup 1.0000x reference)
import jax
import jax.numpy as jnp
from jax.experimental import pallas as pl
from jax.experimental.pallas import tpu as pltpu

B, S, H = 16, 4096, 512


def _body(len_ref, seq_ref, out_ref, scratch, sem_a, sem_b, sem_s):
    copies = []
    for b in range(B):
        row = (len_ref[b] - 1) & (S - 1)
        sem = sem_a if b < 8 else sem_b
        c = pltpu.make_async_copy(seq_ref.at[b, row], scratch.at[b], sem)
        c.start()
        copies.append(c)
    for b in range(8):
        copies[b].wait()
    s1 = pltpu.make_async_copy(
        scratch.at[pl.ds(0, 8)], out_ref.at[pl.ds(0, 8)], sem_s
    )
    s1.start()
    for b in range(8, B):
        copies[b].wait()
    s2 = pltpu.make_async_copy(
        scratch.at[pl.ds(8, 8)], out_ref.at[pl.ds(8, 8)], sem_s
    )
    s2.start()
    s1.wait()
    s2.wait()


def kernel(sequence, lengths):
    return pl.pallas_call(
        _body,
        out_shape=jax.ShapeDtypeStruct((B, H), jnp.float32),
        in_specs=[
            pl.BlockSpec(memory_space=pltpu.MemorySpace.SMEM),
            pl.BlockSpec(memory_space=pl.ANY),
        ],
        out_specs=pl.BlockSpec(memory_space=pl.ANY),
        scratch_shapes=[
            pltpu.VMEM((B, H), jnp.float32),
            pltpu.SemaphoreType.DMA,
            pltpu.SemaphoreType.DMA,
            pltpu.SemaphoreType.DMA,
        ],
    )(lengths.astype(jnp.int32), sequence)


# speedup vs baseline: 1.0123x; 1.0123x over previous
import jax
import jax.numpy as jnp
from jax.experimental import pallas as pl
from jax.experimental.pallas import tpu as pltpu

B, S, H = 16, 4096, 512


def _body(len_ref, seq_ref, out_ref, sem):
    copies = []
    for b in range(B):
        row = (len_ref[b] - 1) & (S - 1)
        c = pltpu.make_async_copy(seq_ref.at[b, row], out_ref.at[b], sem)
        c.start()
        copies.append(c)
    for c in copies:
        c.wait()


def kernel(sequence, lengths):
    return pl.pallas_call(
        _body,
        grid_spec=pltpu.PrefetchScalarGridSpec(
            num_scalar_prefetch=1,
            grid=(1,),
            in_specs=[pl.BlockSpec(memory_space=pl.ANY)],
            out_specs=pl.BlockSpec((B, H), lambda i, idx: (0, 0)),
            scratch_shapes=[pltpu.SemaphoreType.DMA],
        ),
        out_shape=jax.ShapeDtypeStruct((B, H), jnp.float32),
    )(lengths.astype(jnp.int32), sequence)
